# baseline (speedup 1.0000x reference)
import jax
import jax.numpy as jnp
from jax import lax
from jax.experimental import pallas as pl
from jax.experimental.pallas import tpu as pltpu
from jax.experimental.pallas import tpu_sc as plsc

_N = 10000
_E = 320000
_D = 128
_DH = 48
_NP = 3
_H = 256
_NC, _NS, _L = 2, 16, 16
_NW = _NC * _NS
_K = 128
_NB = 84
_EP = _NW * _NB * _K
_NT = 10240
_RPT = _NT // _NS
_DEGW = 16

_f32 = jnp.float32


def _deg_body(dst_hbm, out_hbm, dstv, ones_v, zrow, acc):
    cid = lax.axis_index("c")
    sid = lax.axis_index("s")
    wid = sid * _NC + cid

    pltpu.sync_copy(dst_hbm.at[wid], dstv)

    @pl.loop(0, _K)
    def _fill(i):
        ones_v[i, :] = jnp.ones((_DEGW,), _f32)
        zrow[i, :] = jnp.zeros((_DEGW,), _f32)

    for k in range(_RPT // _K):
        pltpu.sync_copy(zrow, acc.at[pl.ds(sid * _RPT + k * _K, _K)])
    plsc.subcore_barrier()

    @pl.loop(0, _NB)
    def _accum(b):
        pltpu.sync_copy(ones_v, acc.at[dstv.at[b]], add=True)

    plsc.subcore_barrier()
    pltpu.sync_copy(acc.at[pl.ds(sid * _RPT, _RPT)],
                    out_hbm.at[cid, pl.ds(sid * _RPT, _RPT)])


_deg_call = pl.kernel(
    _deg_body,
    out_type=jax.ShapeDtypeStruct((_NC, _NT, _DEGW), _f32),
    mesh=plsc.VectorSubcoreMesh(core_axis_name="c", subcore_axis_name="s",
                                num_cores=_NC, num_subcores=_NS),
    scratch_types=[
        pltpu.VMEM((_NB, _K), jnp.int32),
        pltpu.VMEM((_K, _DEGW), _f32),
        pltpu.VMEM((_K, _DEGW), _f32),
        pltpu.VMEM_SHARED((_NT, _DEGW), _f32),
    ],
    compiler_params=pltpu.CompilerParams(use_tc_tiling_on_sc=False),
)


def _prop_body(u_hbm, src_hbm, dst_hbm, out_hbm,
               srcv, dstv, rb0, rb1, rb2, rb3, tbl, acc,
               g0, g1, g2, g3, c0, c1, c2, c3):
    cid = lax.axis_index("c")
    sid = lax.axis_index("s")
    wid = sid * _NC + cid

    pltpu.sync_copy(src_hbm.at[wid], srcv)
    pltpu.sync_copy(dst_hbm.at[wid], dstv)

    for p in range(_NP):
        pltpu.sync_copy(u_hbm.at[pl.ds(sid * _RPT, _RPT), pl.ds(p * _DH, _DH)],
                        tbl.at[pl.ds(sid * _RPT, _RPT)])

        @pl.loop(0, _K)
        def _zero(i):
            for j in range(_DH // _L):
                rb0[i, pl.ds(j * _L, _L)] = jnp.zeros((_L,), _f32)

        for k in range(_RPT // _K):
            pltpu.sync_copy(rb0, acc.at[pl.ds(sid * _RPT + k * _K, _K)])
        plsc.subcore_barrier()

        bufs = ((rb0, g0, c0), (rb1, g1, c1), (rb2, g2, c2), (rb3, g3, c3))
        for j, (rb, gs, cs) in enumerate(bufs):
            pltpu.async_copy(tbl.at[srcv.at[j]], rb, gs)

        @pl.loop(0, _NB - 4, step=4)
        def _main(b):
            for j, (rb, gs, cs) in enumerate(bufs):
                pltpu.make_async_copy(tbl.at[srcv.at[b + j]], rb, gs).wait()
                pltpu.async_copy(rb, acc.at[dstv.at[b + j]], cs, add=True)
                pltpu.make_async_copy(rb, acc.at[dstv.at[b + j]], cs).wait()
                pltpu.async_copy(tbl.at[srcv.at[b + j + 4]], rb, gs)

        for j, (rb, gs, cs) in enumerate(bufs):
            pltpu.make_async_copy(tbl.at[srcv.at[_NB - 4 + j]], rb, gs).wait()
            pltpu.sync_copy(rb, acc.at[dstv.at[_NB - 4 + j]], add=True)

        plsc.subcore_barrier()
        pltpu.sync_copy(acc.at[pl.ds(sid * _RPT, _RPT)],
                        out_hbm.at[cid, p, pl.ds(sid * _RPT, _RPT)])


_prop_call = pl.kernel(
    _prop_body,
    out_type=jax.ShapeDtypeStruct((_NC, _NP, _NT, _DH), _f32),
    mesh=plsc.VectorSubcoreMesh(core_axis_name="c", subcore_axis_name="s",
                                num_cores=_NC, num_subcores=_NS),
    scratch_types=[
        pltpu.VMEM((_NB, _K), jnp.int32),
        pltpu.VMEM((_NB, _K), jnp.int32),
        pltpu.VMEM((_K, _DH), _f32),
        pltpu.VMEM((_K, _DH), _f32),
        pltpu.VMEM((_K, _DH), _f32),
        pltpu.VMEM((_K, _DH), _f32),
        pltpu.VMEM_SHARED((_NT, _DH), _f32),
        pltpu.VMEM_SHARED((_NT, _DH), _f32),
    ] + [pltpu.SemaphoreType.DMA] * 8,
    compiler_params=pltpu.CompilerParams(use_tc_tiling_on_sc=False),
)


def _split3(u):
    zpad = jnp.zeros((u.shape[0], _NP * _DH - _D), _f32)
    return jnp.concatenate([u, zpad], axis=-1)


def _tc1_body(degs_ref, xp_ref, dinv_ref, u0_ref):
    deg = degs_ref[0] + degs_ref[1] + 1.0
    dinv = lax.rsqrt(deg)
    dinv_ref[...] = dinv
    u1 = xp_ref[...] * dinv
    u0_ref[...] = _split3(u1)


_tc1_call = pl.pallas_call(
    _tc1_body,
    out_shape=[jax.ShapeDtypeStruct((_NT, 1), _f32),
               jax.ShapeDtypeStruct((_NT, _NP * _DH), _f32)],
)

_RB = 2048


def _combine(acc_ref):
    p2 = (acc_ref[0, 2] + acc_ref[1, 2])[:, :_D - 2 * _DH]
    return jnp.concatenate([acc_ref[0, 0] + acc_ref[1, 0],
                            acc_ref[0, 1] + acc_ref[1, 1], p2], axis=-1)


def _tc2_body(acc_ref, xp_ref, dinv_ref, w1_ref, b1_ref, w2_ref,
              g_ref, u0_ref):
    dinv = dinv_ref[...]
    p1 = dinv * _combine(acc_ref) + (dinv * dinv) * xp_ref[...]
    h = jnp.dot(p1, w1_ref[...], preferred_element_type=_f32) + b1_ref[...]
    h = jnp.maximum(h, 0.0)
    g = jnp.dot(h, w2_ref[...], preferred_element_type=_f32)
    g_ref[...] = g
    u2 = g * dinv
    u0_ref[...] = _split3(u2)


_tc2_call = pl.pallas_call(
    _tc2_body,
    grid=(_NT // _RB,),
    in_specs=[
        pl.BlockSpec((_NC, _NP, _RB, _DH), lambda i: (0, 0, i, 0)),
        pl.BlockSpec((_RB, _D), lambda i: (i, 0)),
        pl.BlockSpec((_RB, 1), lambda i: (i, 0)),
        pl.BlockSpec((_D, _H), lambda i: (0, 0)),
        pl.BlockSpec((1, _H), lambda i: (0, 0)),
        pl.BlockSpec((_H, _D), lambda i: (0, 0)),
    ],
    out_specs=[
        pl.BlockSpec((_RB, _D), lambda i: (i, 0)),
        pl.BlockSpec((_RB, _NP * _DH), lambda i: (i, 0)),
    ],
    out_shape=[jax.ShapeDtypeStruct((_NT, _D), _f32),
               jax.ShapeDtypeStruct((_NT, _NP * _DH), _f32)],
)


def _tc3_body(acc_ref, g_ref, dinv_ref, b2_ref, out_ref):
    dinv = dinv_ref[...]
    out_ref[...] = (dinv * _combine(acc_ref)
                    + (dinv * dinv) * g_ref[...] + b2_ref[...])


_tc3_call = pl.pallas_call(
    _tc3_body,
    grid=(_NT // _RB,),
    in_specs=[
        pl.BlockSpec((_NC, _NP, _RB, _DH), lambda i: (0, 0, i, 0)),
        pl.BlockSpec((_RB, _D), lambda i: (i, 0)),
        pl.BlockSpec((_RB, 1), lambda i: (i, 0)),
        pl.BlockSpec((1, _D), lambda i: (0, 0)),
    ],
    out_specs=pl.BlockSpec((_RB, _D), lambda i: (i, 0)),
    out_shape=jax.ShapeDtypeStruct((_NT, _D), _f32),
)


def kernel(x, edge_index, W1, b1, W2, b2):
    src = edge_index[0].astype(jnp.int32)
    dst = edge_index[1].astype(jnp.int32)
    pad = _EP - _E
    srcp = jnp.concatenate([src, jnp.full((pad,), _N, jnp.int32)]
                           ).reshape(_NW, _NB, _K)
    dstp = jnp.concatenate([dst, jnp.full((pad,), _N, jnp.int32)]
                           ).reshape(_NW, _NB, _K)
    xp = jnp.concatenate([x, jnp.zeros((_NT - _N, _D), x.dtype)], axis=0)

    degs = _deg_call(dstp)
    deg1 = degs[:, :, :1]
    dinv, u1t = _tc1_call(deg1, xp)
    acc1 = _prop_call(u1t, srcp, dstp)
    g, u2t = _tc2_call(acc1, xp, dinv, W1, b1.reshape(1, _H), W2)
    acc2 = _prop_call(u2t, srcp, dstp)
    out = _tc3_call(acc2, g, dinv, b2.reshape(1, _D))
    return out[:_N]
